# speedup vs baseline: 1.0171x; 1.0171x over previous
import functools
import math

import jax
import jax.numpy as jnp
from jax import lax
from jax.experimental import pallas as pl
from jax.experimental.pallas import tpu as pltpu
from jax.experimental.pallas import tpu_sc as plsc

D_MODEL = 1024
SCALE = math.sqrt(D_MODEL)
L = 16
NC, NS = 2, 16
NW = NC * NS

CHUNK = 64


def _make_emb(B: int, D: int):
    bpw = B // NW
    nchunk = bpw // CHUNK
    mesh = plsc.VectorSubcoreMesh(core_axis_name="c", subcore_axis_name="s")

    @functools.partial(
        pl.kernel,
        mesh=mesh,
        out_type=jax.ShapeDtypeStruct((B, D), jnp.float32),
        scratch_types=[
            pltpu.VMEM((bpw,), jnp.int32),
            pltpu.VMEM((CHUNK, D), jnp.float32),
            pltpu.SemaphoreType.DMA,
        ],
    )
    def emb(idx_hbm, table_hbm, out_hbm, idx_v, rows_v, sem):
        wid = lax.axis_index("s") * NC + lax.axis_index("c")
        base = wid * bpw
        pltpu.sync_copy(idx_hbm.at[pl.ds(base, bpw)], idx_v)
        for c in range(nchunk):
            pltpu.async_copy(
                table_hbm.at[idx_v.at[pl.ds(c * CHUNK, CHUNK)]], rows_v, sem
            ).wait()

            def body(r, carry):
                for j in range(D // L):
                    sl = pl.ds(j * L, L)
                    rows_v[r, sl] = rows_v[r, sl] * SCALE
                return carry

            lax.fori_loop(0, CHUNK, body, 0)
            pltpu.sync_copy(rows_v, out_hbm.at[pl.ds(base + c * CHUNK, CHUNK)])

    return emb


def kernel(x, table):
    b, s = x.shape
    v, d = table.shape
    idx = x.reshape(-1).astype(jnp.int32)
    out = _make_emb(b * s, d)(idx, table)
    return out.reshape(b, s, d)


# speedup vs baseline: 1.3168x; 1.2947x over previous
import functools
import math

import jax
import jax.numpy as jnp
from jax import lax
from jax.experimental import pallas as pl
from jax.experimental.pallas import tpu as pltpu
from jax.experimental.pallas import tpu_sc as plsc

D_MODEL = 1024
SCALE = math.sqrt(D_MODEL)
L = 16
NC, NS = 2, 16
NW = NC * NS

CHUNK = 32
NBUF = 3


def _make_emb(B: int, D: int):
    bpw = B // NW
    nchunk = bpw // CHUNK
    mesh = plsc.VectorSubcoreMesh(core_axis_name="c", subcore_axis_name="s")

    @functools.partial(
        pl.kernel,
        mesh=mesh,
        out_type=jax.ShapeDtypeStruct((B, D), jnp.float32),
        scratch_types=[
            pltpu.VMEM((bpw,), jnp.int32),
            *[pltpu.VMEM((CHUNK, D), jnp.float32) for _ in range(NBUF)],
            *[pltpu.SemaphoreType.DMA for _ in range(2 * NBUF)],
        ],
    )
    def emb(idx_hbm, table_hbm, out_hbm, idx_v, *rest):
        bufs = rest[:NBUF]
        gsem = rest[NBUF : 2 * NBUF]
        wsem = rest[2 * NBUF :]

        wid = lax.axis_index("s") * NC + lax.axis_index("c")
        base = wid * bpw
        pltpu.sync_copy(idx_hbm.at[pl.ds(base, bpw)], idx_v)

        def gather(c):
            s = c % NBUF
            return pltpu.async_copy(
                table_hbm.at[idx_v.at[pl.ds(c * CHUNK, CHUNK)]], bufs[s], gsem[s]
            )

        def scale(s):
            def body(r, carry):
                for j in range(D // L):
                    sl = pl.ds(j * L, L)
                    bufs[s][r, sl] = bufs[s][r, sl] * SCALE
                return carry

            lax.fori_loop(0, CHUNK, body, 0)

        gd = [None] * nchunk
        wd = [None] * nchunk
        for c in range(NBUF):
            gd[c] = gather(c)
        for c in range(nchunk):
            s = c % NBUF
            if c >= 1 and c + NBUF - 1 < nchunk:
                wd[c - 1].wait()
                gd[c + NBUF - 1] = gather(c + NBUF - 1)
            gd[c].wait()
            scale(s)
            wd[c] = pltpu.async_copy(
                bufs[s], out_hbm.at[pl.ds(base + c * CHUNK, CHUNK)], wsem[s]
            )
        for c in range(nchunk):
            if c >= nchunk - NBUF:
                wd[c].wait()

    return emb


def kernel(x, table):
    b, s = x.shape
    v, d = table.shape
    idx = x.reshape(-1).astype(jnp.int32)
    out = _make_emb(b * s, d)(idx, table)
    return out.reshape(b, s, d)


# speedup vs baseline: 1.4641x; 1.1118x over previous
import functools
import math

import jax
import jax.numpy as jnp
from jax import lax
from jax.experimental import pallas as pl
from jax.experimental.pallas import tpu as pltpu
from jax.experimental.pallas import tpu_sc as plsc

D_MODEL = 1024
SCALE = math.sqrt(D_MODEL)
L = 16
NC, NS = 2, 16
NW = NC * NS

CHUNK = 16
NBUF = 7
LEAD = 5


def _make_emb(B: int, D: int):
    bpw = B // NW
    nchunk = bpw // CHUNK
    mesh = plsc.VectorSubcoreMesh(core_axis_name="c", subcore_axis_name="s")

    @functools.partial(
        pl.kernel,
        mesh=mesh,
        out_type=jax.ShapeDtypeStruct((B, D), jnp.float32),
        scratch_types=[
            pltpu.VMEM((bpw,), jnp.int32),
            *[pltpu.VMEM((CHUNK, D), jnp.float32) for _ in range(NBUF)],
            *[pltpu.SemaphoreType.DMA for _ in range(2 * NBUF)],
        ],
    )
    def emb(idx_hbm, table_hbm, out_hbm, idx_v, *rest):
        bufs = rest[:NBUF]
        gsem = rest[NBUF : 2 * NBUF]
        wsem = rest[2 * NBUF :]

        wid = lax.axis_index("s") * NC + lax.axis_index("c")
        base = wid * bpw
        pltpu.sync_copy(idx_hbm.at[pl.ds(base, bpw)], idx_v)

        def gather(c):
            s = c % NBUF
            return pltpu.async_copy(
                table_hbm.at[idx_v.at[pl.ds(c * CHUNK, CHUNK)]], bufs[s], gsem[s]
            )

        def scale(s):
            def body(r, carry):
                for j in range(D // L):
                    sl = pl.ds(j * L, L)
                    bufs[s][r, sl] = bufs[s][r, sl] * SCALE
                return carry

            lax.fori_loop(0, CHUNK, body, 0)

        gd = [None] * nchunk
        wd = [None] * nchunk
        w_waited = [False] * nchunk
        for c in range(min(LEAD, nchunk)):
            gd[c] = gather(c)
        for c in range(nchunk):
            s = c % NBUF
            nxt = c + LEAD
            if nxt < nchunk:
                prev = nxt - NBUF
                if prev >= 0:
                    wd[prev].wait()
                    w_waited[prev] = True
                gd[nxt] = gather(nxt)
            gd[c].wait()
            scale(s)
            wd[c] = pltpu.async_copy(
                bufs[s], out_hbm.at[pl.ds(base + c * CHUNK, CHUNK)], wsem[s]
            )
        for c in range(nchunk):
            if not w_waited[c]:
                wd[c].wait()

    return emb


def kernel(x, table):
    b, s = x.shape
    v, d = table.shape
    idx = x.reshape(-1).astype(jnp.int32)
    out = _make_emb(b * s, d)(idx, table)
    return out.reshape(b, s, d)
